# Initial kernel scaffold; baseline (speedup 1.0000x reference)
import functools

import jax
import jax.numpy as jnp
from jax import lax
from jax.experimental import pallas as pl
from jax.experimental.pallas import tpu as pltpu
from jax.experimental.pallas import tpu_sc as plsc

N_NODES = 10000
N_EDGES = 320000
D = 128
NUM_SC = 2
NUM_TILES = 16
NUM_WORKERS = NUM_SC * NUM_TILES
E_PER_TILE = N_EDGES // NUM_WORKERS
CHUNK = 80
NCHUNK = E_PER_TILE // CHUNK
ROWS_PER_TILE = N_NODES // NUM_TILES

_mesh = plsc.VectorSubcoreMesh(
    core_axis_name="c", subcore_axis_name="s",
    num_cores=NUM_SC, num_subcores=NUM_TILES,
)


@functools.partial(
    pl.kernel,
    out_type=jax.ShapeDtypeStruct((NUM_SC, N_NODES, D), jnp.float32),
    mesh=_mesh,
    scratch_types=[
        pltpu.VMEM((E_PER_TILE,), jnp.int32),
        pltpu.VMEM((E_PER_TILE,), jnp.float32),
        pltpu.VMEM((NCHUNK, CHUNK), jnp.int32),
        pltpu.VMEM((CHUNK, D), jnp.float32),
        pltpu.VMEM_SHARED((N_NODES, D), jnp.float32),
        pltpu.SemaphoreType.DMA,
    ],
)
def _spmm_sc(x_hbm, col_hbm, val_hbm, row_hbm, zero_hbm, out_hbm,
             colv, valv, rowv, msg, acc, sem):
    c = lax.axis_index("c")
    s = lax.axis_index("s")
    wid = c * NUM_TILES + s
    base = wid * E_PER_TILE

    pltpu.sync_copy(zero_hbm, acc.at[pl.ds(s * ROWS_PER_TILE, ROWS_PER_TILE)])
    pltpu.sync_copy(col_hbm.at[pl.ds(base, E_PER_TILE)], colv)
    pltpu.sync_copy(val_hbm.at[pl.ds(base, E_PER_TILE)], valv)
    pltpu.sync_copy(row_hbm.at[wid], rowv)
    plsc.subcore_barrier()

    @pl.loop(0, NCHUNK)
    def _chunk(j):
        pltpu.async_copy(
            x_hbm.at[colv.at[pl.ds(j * CHUNK, CHUNK)]], msg, sem).wait()

        @pl.loop(0, CHUNK)
        def _row(r):
            vb = plsc.load_gather(
                valv, [jnp.full((16,), j * CHUNK + r, jnp.int32)])
            for q in range(D // 16):
                msg[r, pl.ds(q * 16, 16)] = msg[r, pl.ds(q * 16, 16)] * vb

        pltpu.sync_copy(msg, acc.at[rowv.at[j]], add=True)

    plsc.subcore_barrier()
    pltpu.sync_copy(acc.at[pl.ds(s * ROWS_PER_TILE, ROWS_PER_TILE)],
                    out_hbm.at[c, pl.ds(s * ROWS_PER_TILE, ROWS_PER_TILE)])


def _combine_tc(p_ref, w_ref, b_ref, o_ref):
    agg = p_ref[0] + p_ref[1]
    o_ref[...] = (
        jnp.dot(agg, w_ref[...], preferred_element_type=jnp.float32)
        + b_ref[...]
    )


_BLK_M = 2000


def kernel(x, adj_mat_indices, adj_mat_values, weights, bias):
    row = adj_mat_indices[0].reshape(NUM_WORKERS, NCHUNK, CHUNK)
    col = adj_mat_indices[1]
    zero = jnp.zeros((ROWS_PER_TILE, D), jnp.float32)
    parts = _spmm_sc(x, col, adj_mat_values, row, zero)
    return pl.pallas_call(
        _combine_tc,
        grid=(N_NODES // _BLK_M,),
        in_specs=[
            pl.BlockSpec((NUM_SC, _BLK_M, D), lambda i: (0, i, 0)),
            pl.BlockSpec((D, D), lambda i: (0, 0)),
            pl.BlockSpec((1, D), lambda i: (0, 0)),
        ],
        out_specs=pl.BlockSpec((_BLK_M, D), lambda i: (i, 0)),
        out_shape=jax.ShapeDtypeStruct((N_NODES, D), jnp.float32),
    )(parts, weights, bias.reshape(1, D))


# speedup vs baseline: 6.7792x; 6.7792x over previous
import functools

import jax
import jax.numpy as jnp
from jax import lax
from jax.experimental import pallas as pl
from jax.experimental.pallas import tpu as pltpu
from jax.experimental.pallas import tpu_sc as plsc

N_NODES = 10000
N_EDGES = 320000
D = 128
NUM_SC = 2
NUM_TILES = 16
NUM_WORKERS = NUM_SC * NUM_TILES
E_PER_TILE = N_EDGES // NUM_WORKERS
CHUNK = 80
NCHUNK = E_PER_TILE // CHUNK
N_PAD = 10240
ROWS_PER_TILE = N_PAD // NUM_TILES

_mesh = plsc.VectorSubcoreMesh(
    core_axis_name="c", subcore_axis_name="s",
    num_cores=NUM_SC, num_subcores=NUM_TILES,
)


@functools.partial(
    pl.kernel,
    out_type=jax.ShapeDtypeStruct((NUM_SC, N_PAD, D), jnp.float32),
    mesh=_mesh,
    scratch_types=[
        pltpu.VMEM((E_PER_TILE,), jnp.int32),
        pltpu.VMEM((E_PER_TILE,), jnp.float32),
        pltpu.VMEM((NCHUNK, CHUNK), jnp.int32),
        pltpu.VMEM((CHUNK, D), jnp.float32),
        pltpu.VMEM_SHARED((N_PAD, D), jnp.float32),
        pltpu.SemaphoreType.DMA,
    ],
)
def _spmm_sc(x_hbm, col_hbm, val_hbm, row_hbm, zero_hbm, out_hbm,
             colv, valv, rowv, msg, acc, sem):
    c = lax.axis_index("c")
    s = lax.axis_index("s")
    wid = c * NUM_TILES + s
    base = wid * E_PER_TILE

    pltpu.sync_copy(zero_hbm, acc.at[pl.ds(s * ROWS_PER_TILE, ROWS_PER_TILE)])
    pltpu.sync_copy(col_hbm.at[pl.ds(base, E_PER_TILE)], colv)
    pltpu.sync_copy(val_hbm.at[pl.ds(base, E_PER_TILE)], valv)
    pltpu.sync_copy(row_hbm.at[wid], rowv)
    plsc.subcore_barrier()

    @pl.loop(0, NCHUNK)
    def _chunk(j):
        pltpu.async_copy(
            x_hbm.at[colv.at[pl.ds(j * CHUNK, CHUNK)]], msg, sem).wait()

        @pl.loop(0, CHUNK // 16)
        def _grp(g):
            vv = valv[pl.ds(j * CHUNK + g * 16, 16)]
            for r2 in range(16):
                vb = vv.at[jnp.full((16,), r2, jnp.int32)].get(
                    mode="promise_in_bounds")
                r = g * 16 + r2
                for q in range(D // 16):
                    msg[r, pl.ds(q * 16, 16)] = msg[r, pl.ds(q * 16, 16)] * vb

        pltpu.sync_copy(msg, acc.at[rowv.at[j]], add=True)

    plsc.subcore_barrier()
    pltpu.sync_copy(acc.at[pl.ds(s * ROWS_PER_TILE, ROWS_PER_TILE)],
                    out_hbm.at[c, pl.ds(s * ROWS_PER_TILE, ROWS_PER_TILE)])


def _combine_tc(p_ref, w_ref, b_ref, o_ref):
    agg = p_ref[0] + p_ref[1]
    o_ref[...] = (
        jnp.dot(agg, w_ref[...], preferred_element_type=jnp.float32)
        + b_ref[...]
    )


_BLK_M = 2000


def kernel(x, adj_mat_indices, adj_mat_values, weights, bias):
    row = adj_mat_indices[0].reshape(NUM_WORKERS, NCHUNK, CHUNK)
    col = adj_mat_indices[1]
    zero = jnp.zeros((ROWS_PER_TILE, D), jnp.float32)
    parts = _spmm_sc(x, col, adj_mat_values, row, zero)
    return pl.pallas_call(
        _combine_tc,
        grid=(N_NODES // _BLK_M,),
        in_specs=[
            pl.BlockSpec((NUM_SC, _BLK_M, D), lambda i: (0, i, 0)),
            pl.BlockSpec((D, D), lambda i: (0, 0)),
            pl.BlockSpec((1, D), lambda i: (0, 0)),
        ],
        out_specs=pl.BlockSpec((_BLK_M, D), lambda i: (i, 0)),
        out_shape=jax.ShapeDtypeStruct((N_NODES, D), jnp.float32),
    )(parts, weights, bias.reshape(1, D))


# speedup vs baseline: 7.1814x; 1.0593x over previous
import functools

import jax
import jax.numpy as jnp
from jax import lax
from jax.experimental import pallas as pl
from jax.experimental.pallas import tpu as pltpu
from jax.experimental.pallas import tpu_sc as plsc

N_NODES = 10000
N_EDGES = 320000
D = 128
NUM_SC = 2
NUM_TILES = 16
NUM_WORKERS = NUM_SC * NUM_TILES
E_PER_TILE = N_EDGES // NUM_WORKERS
CHUNK = 80
NCHUNK = 126
E_PAD = NCHUNK * CHUNK
N_PAD = 10240
ROWS_PER_TILE = N_PAD // NUM_TILES

_mesh = plsc.VectorSubcoreMesh(
    core_axis_name="c", subcore_axis_name="s",
    num_cores=NUM_SC, num_subcores=NUM_TILES,
)


@functools.partial(
    pl.kernel,
    out_type=jax.ShapeDtypeStruct((NUM_SC, N_PAD, D), jnp.float32),
    mesh=_mesh,
    scratch_types=[
        [pltpu.VMEM((3, CHUNK), jnp.int32) for _ in range(6)],
        [pltpu.VMEM((CHUNK, D), jnp.float32) for _ in range(3)],
        pltpu.VMEM_SHARED((N_PAD, D), jnp.float32),
        [pltpu.SemaphoreType.DMA for _ in range(6)],
        [pltpu.SemaphoreType.DMA for _ in range(3)],
        [pltpu.SemaphoreType.DMA for _ in range(3)],
    ],
)
def _spmm_sc(x_hbm, pk_hbm, zero_hbm, out_hbm,
             pbufs, msgs, acc, esems, gsems, ssems):
    c = lax.axis_index("c")
    s = lax.axis_index("s")
    wid = c * NUM_TILES + s

    pltpu.sync_copy(zero_hbm, acc.at[pl.ds(s * ROWS_PER_TILE, ROWS_PER_TILE)])
    plsc.subcore_barrier()

    def issue_edges(j, p):
        pltpu.async_copy(pk_hbm.at[wid, j], pbufs[p], esems[p])

    def wait_edges(p):
        pltpu.make_async_copy(pk_hbm.at[wid, 0], pbufs[p], esems[p]).wait()

    def issue_gather(k, p):
        pltpu.async_copy(x_hbm.at[pbufs[p].at[1]], msgs[k], gsems[k])

    def wait_gather(k, p):
        pltpu.make_async_copy(
            x_hbm.at[pbufs[p].at[1]], msgs[k], gsems[k]).wait()

    def issue_scatter(k, p):
        pltpu.async_copy(msgs[k], acc.at[pbufs[p].at[0]], ssems[k], add=True)

    def wait_scatter(k, p):
        pltpu.make_async_copy(msgs[k], acc.at[pbufs[p].at[0]], ssems[k]).wait()

    def scale(k, p):
        buf = msgs[k]

        @pl.loop(0, CHUNK // 16)
        def _grp(g):
            vv = lax.bitcast_convert_type(
                pbufs[p][2, pl.ds(g * 16, 16)], jnp.float32)

            @pl.loop(0, 16, unroll=4)
            def _row(r2):
                vb = vv.at[jnp.full((16,), r2, jnp.int32)].get(
                    mode="promise_in_bounds")
                r = g * 16 + r2
                for q in range(D // 16):
                    buf[r, pl.ds(q * 16, 16)] = buf[r, pl.ds(q * 16, 16)] * vb

    def chunk_body(j, jph, *, ws=True, ie=True, ig=True):
        k, p = jph % 3, jph % 6
        kn, pn = (jph + 1) % 3, (jph + 1) % 6
        if ws:
            wait_scatter(kn, (jph + 4) % 6)
        if ie:
            issue_edges(j + 3, (jph + 3) % 6)
        if ig:
            wait_edges(pn)
            issue_gather(kn, pn)
        wait_gather(k, p)
        scale(k, p)
        issue_scatter(k, p)

    issue_edges(0, 0)
    issue_edges(1, 1)
    issue_edges(2, 2)
    wait_edges(0)
    issue_gather(0, 0)
    chunk_body(0, 0, ws=False)
    chunk_body(1, 1, ws=False)
    chunk_body(2, 2)

    @pl.loop(3, NCHUNK - 3, step=6)
    def _six(J):
        for t in range(6):
            chunk_body(J + t, 3 + t)

    chunk_body(NCHUNK - 3, NCHUNK - 3, ie=False)
    chunk_body(NCHUNK - 2, NCHUNK - 2, ie=False)
    chunk_body(NCHUNK - 1, NCHUNK - 1, ie=False, ig=False)
    wait_scatter((NCHUNK - 2) % 3, (NCHUNK - 2) % 6)
    wait_scatter((NCHUNK - 1) % 3, (NCHUNK - 1) % 6)

    plsc.subcore_barrier()
    pltpu.sync_copy(acc.at[pl.ds(s * ROWS_PER_TILE, ROWS_PER_TILE)],
                    out_hbm.at[c, pl.ds(s * ROWS_PER_TILE, ROWS_PER_TILE)])


def _combine_tc(p_ref, w_ref, b_ref, o_ref):
    agg = p_ref[0] + p_ref[1]
    o_ref[...] = (
        jnp.dot(agg, w_ref[...], preferred_element_type=jnp.float32)
        + b_ref[...]
    )


_BLK_M = 2000


def kernel(x, adj_mat_indices, adj_mat_values, weights, bias):
    pad = E_PAD - E_PER_TILE
    row = jnp.pad(adj_mat_indices[0].reshape(NUM_WORKERS, E_PER_TILE),
                  ((0, 0), (0, pad))).reshape(NUM_WORKERS, NCHUNK, CHUNK)
    col = jnp.pad(adj_mat_indices[1].reshape(NUM_WORKERS, E_PER_TILE),
                  ((0, 0), (0, pad))).reshape(NUM_WORKERS, NCHUNK, CHUNK)
    val = jnp.pad(
        lax.bitcast_convert_type(adj_mat_values, jnp.int32).reshape(
            NUM_WORKERS, E_PER_TILE),
        ((0, 0), (0, pad))).reshape(NUM_WORKERS, NCHUNK, CHUNK)
    pk = jnp.stack([row, col, val], axis=2)
    zero = jnp.zeros((ROWS_PER_TILE, D), jnp.float32)
    parts = _spmm_sc(x, pk, zero)
    return pl.pallas_call(
        _combine_tc,
        grid=(N_NODES // _BLK_M,),
        in_specs=[
            pl.BlockSpec((NUM_SC, _BLK_M, D), lambda i: (0, i, 0)),
            pl.BlockSpec((D, D), lambda i: (0, 0)),
            pl.BlockSpec((1, D), lambda i: (0, 0)),
        ],
        out_specs=pl.BlockSpec((_BLK_M, D), lambda i: (i, 0)),
        out_shape=jax.ShapeDtypeStruct((N_NODES, D), jnp.float32),
    )(parts, weights, bias.reshape(1, D))


# speedup vs baseline: 12.5356x; 1.7456x over previous
import functools

import jax
import jax.numpy as jnp
from jax import lax
from jax.experimental import pallas as pl
from jax.experimental.pallas import tpu as pltpu
from jax.experimental.pallas import tpu_sc as plsc

N_NODES = 10000
N_EDGES = 320000
D = 128
NUM_SC = 2
NUM_TILES = 16
NUM_WORKERS = NUM_SC * NUM_TILES
E_PER_TILE = N_EDGES // NUM_WORKERS
CHUNK = 80
NCHUNK = E_PER_TILE // CHUNK
N_PAD = 10240
ROWS_PER_TILE = N_PAD // NUM_TILES

_mesh = plsc.VectorSubcoreMesh(
    core_axis_name="c", subcore_axis_name="s",
    num_cores=NUM_SC, num_subcores=NUM_TILES,
)


@functools.partial(
    pl.kernel,
    out_type=jax.ShapeDtypeStruct((NUM_SC, N_PAD, D), jnp.float32),
    mesh=_mesh,
    scratch_types=[
        [pltpu.VMEM((CHUNK,), jnp.int32) for _ in range(6)],
        [pltpu.VMEM((CHUNK,), jnp.int32) for _ in range(6)],
        [pltpu.VMEM((CHUNK,), jnp.float32) for _ in range(6)],
        [pltpu.VMEM((CHUNK, D), jnp.float32) for _ in range(3)],
        pltpu.VMEM_SHARED((N_PAD, D), jnp.float32),
        [pltpu.SemaphoreType.DMA for _ in range(6)],
        [pltpu.SemaphoreType.DMA for _ in range(3)],
        [pltpu.SemaphoreType.DMA for _ in range(3)],
    ],
)
def _spmm_sc(x_hbm, row_hbm, col_hbm, val_hbm, zero_hbm, out_hbm,
             rbufs, cbufs, vbufs, fbufs, acc, esems, gsems, ssems):
    c = lax.axis_index("c")
    s = lax.axis_index("s")
    wid = c * NUM_TILES + s
    base = wid * E_PER_TILE

    pltpu.sync_copy(zero_hbm, acc.at[pl.ds(s * ROWS_PER_TILE, ROWS_PER_TILE)])
    plsc.subcore_barrier()

    def issue_edges(j, p):
        sl = pl.ds(base + j * CHUNK, CHUNK)
        pltpu.async_copy(row_hbm.at[sl], rbufs[p], esems[p])
        pltpu.async_copy(col_hbm.at[sl], cbufs[p], esems[p])
        pltpu.async_copy(val_hbm.at[sl], vbufs[p], esems[p])

    def wait_edges(p):
        sl = pl.ds(base, CHUNK)
        pltpu.make_async_copy(row_hbm.at[sl], rbufs[p], esems[p]).wait()
        pltpu.make_async_copy(col_hbm.at[sl], cbufs[p], esems[p]).wait()
        pltpu.make_async_copy(val_hbm.at[sl], vbufs[p], esems[p]).wait()

    def issue_gather(k3, p):
        pltpu.async_copy(x_hbm.at[cbufs[p]], fbufs[k3], gsems[k3])

    def wait_gather(k3, p):
        pltpu.make_async_copy(x_hbm.at[cbufs[p]], fbufs[k3], gsems[k3]).wait()

    def issue_scatter(k3, p):
        pltpu.async_copy(fbufs[k3], acc.at[rbufs[p]], ssems[k3], add=True)

    def wait_scatter(k3, p):
        pltpu.make_async_copy(fbufs[k3], acc.at[rbufs[p]], ssems[k3]).wait()

    def scale(k3, p):
        buf, vals = fbufs[k3], vbufs[p]

        @pl.loop(0, CHUNK // 16)
        def _grp(g):
            vv = vals[pl.ds(g * 16, 16)]

            @pl.loop(0, 16, unroll=4)
            def _row(r2):
                vb = vv.at[jnp.full((16,), r2, jnp.int32)].get(
                    mode="promise_in_bounds")
                r = g * 16 + r2
                for q in range(D // 16):
                    buf[r, pl.ds(q * 16, 16)] = buf[r, pl.ds(q * 16, 16)] * vb

    def chunk_body(j, jph, *, ws=True, ie=True, ig=True):
        k3, p = jph % 3, jph % 6
        if ws:
            wait_scatter((jph + 1) % 3, (jph + 4) % 6)
        if ie:
            issue_edges(j + 3, (jph + 3) % 6)
        if ig:
            wait_edges((jph + 1) % 6)
            issue_gather((jph + 1) % 3, (jph + 1) % 6)
        wait_gather(k3, p)
        scale(k3, p)
        issue_scatter(k3, p)

    issue_edges(0, 0)
    issue_edges(1, 1)
    issue_edges(2, 2)
    wait_edges(0)
    issue_gather(0, 0)
    chunk_body(0, 0, ws=False)
    chunk_body(1, 1, ws=False)
    chunk_body(2, 2)

    @pl.loop(3, NCHUNK - 2, step=6)
    def _six(J):
        for t in range(6):
            chunk_body(J + t, 3 + t)

    chunk_body(NCHUNK - 2, NCHUNK - 2, ie=False)
    chunk_body(NCHUNK - 1, NCHUNK - 1, ie=False, ig=False)
    wait_scatter((NCHUNK - 2) % 3, (NCHUNK - 2) % 6)
    wait_scatter((NCHUNK - 1) % 3, (NCHUNK - 1) % 6)

    plsc.subcore_barrier()
    pltpu.sync_copy(acc.at[pl.ds(s * ROWS_PER_TILE, ROWS_PER_TILE)],
                    out_hbm.at[c, pl.ds(s * ROWS_PER_TILE, ROWS_PER_TILE)])


def _combine_tc(p_ref, w_ref, b_ref, o_ref):
    agg = p_ref[0] + p_ref[1]
    o_ref[...] = (
        jnp.dot(agg, w_ref[...], preferred_element_type=jnp.float32)
        + b_ref[...]
    )


_BLK_M = 2000


def kernel(x, adj_mat_indices, adj_mat_values, weights, bias):
    zero = jnp.zeros((ROWS_PER_TILE, D), jnp.float32)
    parts = _spmm_sc(x, adj_mat_indices[0], adj_mat_indices[1],
                     adj_mat_values, zero)
    return pl.pallas_call(
        _combine_tc,
        grid=(N_NODES // _BLK_M,),
        in_specs=[
            pl.BlockSpec((NUM_SC, _BLK_M, D), lambda i: (0, i, 0)),
            pl.BlockSpec((D, D), lambda i: (0, 0)),
            pl.BlockSpec((1, D), lambda i: (0, 0)),
        ],
        out_specs=pl.BlockSpec((_BLK_M, D), lambda i: (i, 0)),
        out_shape=jax.ShapeDtypeStruct((N_NODES, D), jnp.float32),
    )(parts, weights, bias.reshape(1, D))
